# baseline (speedup 1.0000x reference)
import functools

import jax
import jax.numpy as jnp
from jax import lax
from jax.experimental import pallas as pl
from jax.experimental.pallas import tpu as pltpu
from jax.experimental.pallas import tpu_sc as plsc

_B, _D, _T = 8, 256, 1024
_K = 8192
_TK = 1024
_KT = _K // _TK
_RB = _TK // 8
_NTOK = _B * _T
_LOSS_SCALE = 1.25 / float(_B * _D * _T)

_NC, _NS = 2, 16
_NW = _NC * _NS
_BPW = _NTOK // _NW


def _dist_argmin_body(zb_ref, z2_ref, cb_ref, c2_ref, idx_ref, loss_ref,
                      best_ref, brb_ref, h0v_ref, h0i_ref, acc_ref):
    b = pl.program_id(0)
    kt = pl.program_id(1)

    @pl.when(jnp.logical_or(kt == 0, kt == _KT // 2))
    def _init():
        best_ref[...] = jnp.full((8, _T), jnp.inf, dtype=jnp.float32)
        brb_ref[...] = jnp.zeros((8, _T), dtype=jnp.int32)

    zb16 = (zb_ref[0] * -2.0).astype(jnp.bfloat16)
    cb16 = cb_ref[...].astype(jnp.bfloat16)
    dot = jnp.dot(cb16, zb16, preferred_element_type=jnp.float32)
    s = (z2_ref[0] + dot) + c2_ref[...]

    bestv = best_ref[...]
    besti = brb_ref[...]
    base_rb = kt * _RB
    for r in range(_RB):
        srow = s[r * 8:(r + 1) * 8, :]
        cmp = srow < bestv
        bestv = jnp.where(cmp, srow, bestv)
        besti = jnp.where(cmp, base_rb + r, besti)
    best_ref[...] = bestv
    brb_ref[...] = besti

    def _half_min(vals, rbs):
        kglob = rbs * 8 + lax.broadcasted_iota(jnp.int32, (8, _T), 0)
        m = jnp.min(vals, axis=0, keepdims=True)
        cand = jnp.where(vals == m, kglob, _K)
        return m, jnp.min(cand, axis=0, keepdims=True)

    @pl.when(kt == _KT // 2 - 1)
    def _end_half0():
        m0, i0 = _half_min(best_ref[...], brb_ref[...])
        h0v_ref[...] = m0
        h0i_ref[...] = i0

    @pl.when(kt == _KT - 1)
    def _finalize():
        m1, i1 = _half_min(best_ref[...], brb_ref[...])
        m0 = h0v_ref[...]
        i0 = h0i_ref[...]
        m0q = m0.astype(jnp.bfloat16).astype(jnp.float32)
        take0 = jnp.logical_or(m0q < m1,
                               jnp.logical_and(m0q == m1, i0 < i1))
        idx_ref[0, 0, :] = jnp.where(take0, i0, i1)[0]
        part = jnp.sum(jnp.where(take0, m0, m1))
        tot = jnp.where(b == 0, 0.0, acc_ref[0]) + part
        acc_ref[0] = tot

        @pl.when(b == _B - 1)
        def _loss():
            loss_ref[0, 0] = tot * _LOSS_SCALE


def _dist_argmin(zm2b, z2, cbb, c2):
    return pl.pallas_call(
        _dist_argmin_body,
        grid=(_B, _KT),
        in_specs=[
            pl.BlockSpec((1, _D, _T), lambda b, kt: (b, 0, 0)),
            pl.BlockSpec((1, 1, _T), lambda b, kt: (b, 0, 0)),
            pl.BlockSpec((_TK, _D), lambda b, kt: (kt, 0)),
            pl.BlockSpec((_TK, 1), lambda b, kt: (kt, 0)),
        ],
        out_specs=[
            pl.BlockSpec((1, 1, _T), lambda b, kt: (b, 0, 0)),
            pl.BlockSpec(memory_space=pltpu.SMEM),
        ],
        out_shape=[
            jax.ShapeDtypeStruct((_B, 1, _T), jnp.int32),
            jax.ShapeDtypeStruct((1, 1), jnp.float32),
        ],
        scratch_shapes=[
            pltpu.VMEM((8, _T), jnp.float32),
            pltpu.VMEM((8, _T), jnp.int32),
            pltpu.VMEM((1, _T), jnp.float32),
            pltpu.VMEM((1, _T), jnp.int32),
            pltpu.SMEM((1,), jnp.float32),
        ],
    )(zm2b, z2, cbb, c2)


@functools.lru_cache(maxsize=1)
def _sc_gather_kernel():
    @functools.partial(
        pl.kernel,
        out_type=jax.ShapeDtypeStruct((_NTOK, _D), jnp.float32),
        mesh=plsc.VectorSubcoreMesh(core_axis_name="c", subcore_axis_name="s",
                                    num_cores=_NC, num_subcores=_NS),
        scratch_types=[
            pltpu.VMEM((_BPW,), jnp.int32),
            pltpu.VMEM((_BPW, _D), jnp.float32),
            pltpu.SemaphoreType.DMA,
        ],
    )
    def _sc_gather(cb_hbm, idx_hbm, out_hbm, idx_v, rows_v, sem):
        wid = lax.axis_index("s") * _NC + lax.axis_index("c")
        base = wid * _BPW
        pltpu.sync_copy(idx_hbm.at[pl.ds(base, _BPW)], idx_v)
        pltpu.async_copy(cb_hbm.at[idx_v], rows_v, sem).wait()
        pltpu.sync_copy(rows_v, out_hbm.at[pl.ds(base, _BPW)])

    return _sc_gather


def kernel(z, codebook):
    z_perm = jnp.transpose(z, (0, 2, 1)).reshape(_NTOK, _D)
    z2 = jnp.sum(z_perm ** 2, axis=1, keepdims=True).reshape(_B, 1, _T)
    c2 = jnp.sum(codebook ** 2, axis=1).reshape(_K, 1)

    idx3, loss = _dist_argmin(z, z2, codebook, c2)

    zq_flat = _sc_gather_kernel()(codebook, idx3.reshape(_NTOK))
    z_q = zq_flat.reshape(_B, _T, _D).transpose(0, 2, 1)
    return z_q, idx3.reshape(_B, _T), loss.reshape(())


# speedup vs baseline: 1.2266x; 1.2266x over previous
import functools

import jax
import jax.numpy as jnp
from jax import lax
from jax.experimental import pallas as pl
from jax.experimental.pallas import tpu as pltpu
from jax.experimental.pallas import tpu_sc as plsc

_B, _D, _T = 8, 256, 1024
_K = 8192
_TK = 4096
_KT = _K // _TK
_RB = _TK // 8
_NTOK = _B * _T
_LOSS_SCALE = 1.25 / float(_B * _D * _T)

_NC, _NS = 2, 16
_NW = _NC * _NS
_BPW = _NTOK // _NW


def _dist_argmin_body(zb_ref, z2_ref, cb_ref, c2_ref, idx_ref, loss_ref,
                      best_ref, brb_ref, h0v_ref, h0i_ref, acc_ref):
    b = pl.program_id(0)
    kt = pl.program_id(1)

    @pl.when(jnp.logical_or(kt == 0, kt == _KT // 2))
    def _init():
        best_ref[...] = jnp.full((8, _T), jnp.inf, dtype=jnp.float32)
        brb_ref[...] = jnp.zeros((8, _T), dtype=jnp.int32)

    dot = jnp.dot(cb_ref[...], zb_ref[0],
                  preferred_element_type=jnp.float32)
    s = (z2_ref[0] + dot) + c2_ref[...]

    bestv = best_ref[...]
    besti = brb_ref[...]
    base_rb = kt * _RB
    for r in range(_RB):
        srow = s[r * 8:(r + 1) * 8, :]
        cmp = srow < bestv
        bestv = jnp.where(cmp, srow, bestv)
        besti = jnp.where(cmp, base_rb + r, besti)
    best_ref[...] = bestv
    brb_ref[...] = besti

    def _half_min(vals, rbs):
        kglob = rbs * 8 + lax.broadcasted_iota(jnp.int32, (8, _T), 0)
        m = jnp.min(vals, axis=0, keepdims=True)
        cand = jnp.where(vals == m, kglob, _K)
        return m, jnp.min(cand, axis=0, keepdims=True)

    @pl.when(kt == _KT // 2 - 1)
    def _end_half0():
        m0, i0 = _half_min(best_ref[...], brb_ref[...])
        h0v_ref[...] = m0
        h0i_ref[...] = i0

    @pl.when(kt == _KT - 1)
    def _finalize():
        m1, i1 = _half_min(best_ref[...], brb_ref[...])
        m0 = h0v_ref[...]
        i0 = h0i_ref[...]
        m0q = m0.astype(jnp.bfloat16).astype(jnp.float32)
        take0 = jnp.logical_or(m0q < m1,
                               jnp.logical_and(m0q == m1, i0 < i1))
        idx_ref[0, 0, :] = jnp.where(take0, i0, i1)[0]
        part = jnp.sum(jnp.where(take0, m0, m1))
        tot = jnp.where(b == 0, 0.0, acc_ref[0]) + part
        acc_ref[0] = tot

        @pl.when(b == _B - 1)
        def _loss():
            loss_ref[0, 0] = tot * _LOSS_SCALE


def _dist_argmin(zm2b, z2, cbb, c2):
    return pl.pallas_call(
        _dist_argmin_body,
        grid=(_B, _KT),
        in_specs=[
            pl.BlockSpec((1, _D, _T), lambda b, kt: (b, 0, 0)),
            pl.BlockSpec((1, 1, _T), lambda b, kt: (b, 0, 0)),
            pl.BlockSpec((_TK, _D), lambda b, kt: (kt, 0)),
            pl.BlockSpec((_TK, 1), lambda b, kt: (kt, 0)),
        ],
        out_specs=[
            pl.BlockSpec((1, 1, _T), lambda b, kt: (b, 0, 0)),
            pl.BlockSpec(memory_space=pltpu.SMEM),
        ],
        out_shape=[
            jax.ShapeDtypeStruct((_B, 1, _T), jnp.int32),
            jax.ShapeDtypeStruct((1, 1), jnp.float32),
        ],
        scratch_shapes=[
            pltpu.VMEM((8, _T), jnp.float32),
            pltpu.VMEM((8, _T), jnp.int32),
            pltpu.VMEM((1, _T), jnp.float32),
            pltpu.VMEM((1, _T), jnp.int32),
            pltpu.SMEM((1,), jnp.float32),
        ],
    )(zm2b, z2, cbb, c2)


@functools.lru_cache(maxsize=1)
def _sc_gather_kernel():
    @functools.partial(
        pl.kernel,
        out_type=jax.ShapeDtypeStruct((_NTOK, _D), jnp.float32),
        mesh=plsc.VectorSubcoreMesh(core_axis_name="c", subcore_axis_name="s",
                                    num_cores=_NC, num_subcores=_NS),
        scratch_types=[
            pltpu.VMEM((_BPW,), jnp.int32),
            pltpu.VMEM((_BPW, _D), jnp.float32),
            pltpu.SemaphoreType.DMA,
        ],
    )
    def _sc_gather(cb_hbm, idx_hbm, out_hbm, idx_v, rows_v, sem):
        wid = lax.axis_index("s") * _NC + lax.axis_index("c")
        base = wid * _BPW
        pltpu.sync_copy(idx_hbm.at[pl.ds(base, _BPW)], idx_v)
        pltpu.async_copy(cb_hbm.at[idx_v], rows_v, sem).wait()
        pltpu.sync_copy(rows_v, out_hbm.at[pl.ds(base, _BPW)])

    return _sc_gather


def kernel(z, codebook):
    zm2b = (-2.0 * z).astype(jnp.bfloat16)
    cbb = codebook.astype(jnp.bfloat16)
    z_perm = jnp.transpose(z, (0, 2, 1)).reshape(_NTOK, _D)
    z2 = jnp.sum(z_perm ** 2, axis=1, keepdims=True).reshape(_B, 1, _T)
    c2 = jnp.sum(codebook ** 2, axis=1).reshape(_K, 1)

    idx3, loss = _dist_argmin(zm2b, z2, cbb, c2)

    zq_flat = _sc_gather_kernel()(codebook, idx3.reshape(_NTOK))
    z_q = zq_flat.reshape(_B, _T, _D).transpose(0, 2, 1)
    return z_q, idx3.reshape(_B, _T), loss.reshape(())


# speedup vs baseline: 1.2272x; 1.0005x over previous
import functools

import jax
import jax.numpy as jnp
from jax import lax
from jax.experimental import pallas as pl
from jax.experimental.pallas import tpu as pltpu
from jax.experimental.pallas import tpu_sc as plsc

_B, _D, _T = 8, 256, 1024
_K = 8192
_TK = 4096
_KT = _K // _TK
_RB = _TK // 8
_NTOK = _B * _T
_LOSS_SCALE = 1.25 / float(_B * _D * _T)

_NC, _NS = 2, 16
_NW = _NC * _NS
_BPW = _NTOK // _NW


def _dist_argmin_body(zb_ref, z2_ref, cb_ref, c2_ref, idx_ref, loss_ref,
                      best_ref, brb_ref, h0v_ref, h0i_ref, acc_ref):
    b = pl.program_id(0)
    kt = pl.program_id(1)

    @pl.when(jnp.logical_or(kt == 0, kt == _KT // 2))
    def _init():
        best_ref[...] = jnp.full((8, _T), jnp.inf, dtype=jnp.float32)
        brb_ref[...] = jnp.zeros((8, _T), dtype=jnp.int32)

    dot = jnp.dot(cb_ref[...], zb_ref[0],
                  preferred_element_type=jnp.float32)
    s = (z2_ref[0] + dot) + c2_ref[...]

    bestv = best_ref[...]
    besti = brb_ref[...]
    base_rb = kt * _RB
    for r in range(_RB):
        srow = s[r * 8:(r + 1) * 8, :]
        cmp = srow < bestv
        bestv = jnp.where(cmp, srow, bestv)
        besti = jnp.where(cmp, base_rb + r, besti)
    best_ref[...] = bestv
    brb_ref[...] = besti

    def _half_min(vals, rbs):
        kglob = rbs * 8 + lax.broadcasted_iota(jnp.int32, (8, _T), 0)
        m = jnp.min(vals, axis=0, keepdims=True)
        cand = jnp.where(vals == m, kglob, _K)
        return m, jnp.min(cand, axis=0, keepdims=True)

    @pl.when(kt == _KT // 2 - 1)
    def _end_half0():
        m0, i0 = _half_min(best_ref[...], brb_ref[...])
        h0v_ref[...] = m0
        h0i_ref[...] = i0

    @pl.when(kt == _KT - 1)
    def _finalize():
        m1, i1 = _half_min(best_ref[...], brb_ref[...])
        m0 = h0v_ref[...]
        i0 = h0i_ref[...]
        m0q = m0.astype(jnp.bfloat16).astype(jnp.float32)
        take0 = jnp.logical_or(m0q < m1,
                               jnp.logical_and(m0q == m1, i0 < i1))
        idx_ref[0, 0, :] = jnp.where(take0, i0, i1)[0]
        part = jnp.sum(jnp.where(take0, m0, m1))
        tot = jnp.where(b == 0, 0.0, acc_ref[0]) + part
        acc_ref[0] = tot

        @pl.when(b == _B - 1)
        def _loss():
            loss_ref[0, 0] = tot * _LOSS_SCALE


def _dist_argmin(zm2b, z2, cbb, c2):
    return pl.pallas_call(
        _dist_argmin_body,
        grid=(_B, _KT),
        in_specs=[
            pl.BlockSpec((1, _D, _T), lambda b, kt: (b, 0, 0)),
            pl.BlockSpec((1, 1, _T), lambda b, kt: (b, 0, 0)),
            pl.BlockSpec((_TK, _D), lambda b, kt: (kt, 0)),
            pl.BlockSpec((_TK, 1), lambda b, kt: (kt, 0)),
        ],
        out_specs=[
            pl.BlockSpec((1, 1, _T), lambda b, kt: (b, 0, 0)),
            pl.BlockSpec(memory_space=pltpu.SMEM),
        ],
        out_shape=[
            jax.ShapeDtypeStruct((_B, 1, _T), jnp.int32),
            jax.ShapeDtypeStruct((1, 1), jnp.float32),
        ],
        scratch_shapes=[
            pltpu.VMEM((8, _T), jnp.float32),
            pltpu.VMEM((8, _T), jnp.int32),
            pltpu.VMEM((1, _T), jnp.float32),
            pltpu.VMEM((1, _T), jnp.int32),
            pltpu.SMEM((1,), jnp.float32),
        ],
    )(zm2b, z2, cbb, c2)


@functools.lru_cache(maxsize=1)
def _sc_gather_kernel():
    @functools.partial(
        pl.kernel,
        out_type=jax.ShapeDtypeStruct((_NTOK, _D), jnp.float32),
        mesh=plsc.VectorSubcoreMesh(core_axis_name="c", subcore_axis_name="s",
                                    num_cores=_NC, num_subcores=_NS),
        scratch_types=[
            pltpu.VMEM((_BPW,), jnp.int32),
            pltpu.VMEM((_BPW, _D), jnp.float32),
            pltpu.SemaphoreType.DMA,
        ],
    )
    def _sc_gather(cb_hbm, idx_hbm, out_hbm, idx_v, rows_v, sem):
        wid = lax.axis_index("s") * _NC + lax.axis_index("c")
        base = wid * _BPW
        pltpu.sync_copy(idx_hbm.at[pl.ds(base, _BPW)], idx_v)
        pltpu.async_copy(cb_hbm.at[idx_v], rows_v, sem).wait()
        pltpu.sync_copy(rows_v, out_hbm.at[pl.ds(base, _BPW)])

    return _sc_gather


def kernel(z, codebook):
    zm2b = (-2.0 * z).astype(jnp.bfloat16)
    cbb = codebook.astype(jnp.bfloat16)
    z2 = jnp.sum(z ** 2, axis=1).reshape(_B, 1, _T)
    c2 = jnp.sum(codebook ** 2, axis=1).reshape(_K, 1)

    idx3, loss = _dist_argmin(zm2b, z2, cbb, c2)

    zq_flat = _sc_gather_kernel()(codebook, idx3.reshape(_NTOK))
    z_q = zq_flat.reshape(_B, _T, _D).transpose(0, 2, 1)
    return z_q, idx3.reshape(_B, _T), loss.reshape(())


# speedup vs baseline: 1.2429x; 1.0128x over previous
import functools

import jax
import jax.numpy as jnp
from jax import lax
from jax.experimental import pallas as pl
from jax.experimental.pallas import tpu as pltpu
from jax.experimental.pallas import tpu_sc as plsc

_B, _D, _T = 8, 256, 1024
_K = 8192
_TK = 4096
_KT = _K // _TK
_RB = _TK // 8
_NTOK = _B * _T
_LOSS_SCALE = 1.25 / float(_B * _D * _T)

_NC, _NS = 2, 16
_NW = _NC * _NS
_BPW = _NTOK // _NW


def _dist_argmin_body(zb_ref, z2_ref, cb_ref, c2_ref, idx_ref, loss_ref,
                      best_ref, brb_ref, h0v_ref, h0i_ref, acc_ref):
    b = pl.program_id(0)
    kt = pl.program_id(1)

    @pl.when(jnp.logical_or(kt == 0, kt == _KT // 2))
    def _init():
        best_ref[...] = jnp.full((8, _T), jnp.inf, dtype=jnp.float32)
        brb_ref[...] = jnp.zeros((8, _T), dtype=jnp.int32)

    dot = jnp.dot(cb_ref[...], zb_ref[0],
                  preferred_element_type=jnp.float32)
    s = (z2_ref[0] + dot) + c2_ref[...]

    bestv = best_ref[...]
    besti = brb_ref[...]
    base_rb = kt * _RB
    for r in range(_RB):
        srow = s[r * 8:(r + 1) * 8, :]
        cmp = srow < bestv
        bestv = jnp.where(cmp, srow, bestv)
        besti = jnp.where(cmp, base_rb + r, besti)
    best_ref[...] = bestv
    brb_ref[...] = besti

    def _half_min(vals, rbs):
        kglob = rbs * 8 + lax.broadcasted_iota(jnp.int32, (8, _T), 0)
        m = jnp.min(vals, axis=0, keepdims=True)
        cand = jnp.where(vals == m, kglob, _K)
        return m, jnp.min(cand, axis=0, keepdims=True)

    @pl.when(kt == _KT // 2 - 1)
    def _end_half0():
        m0, i0 = _half_min(best_ref[...], brb_ref[...])
        h0v_ref[...] = m0
        h0i_ref[...] = i0

    @pl.when(kt == _KT - 1)
    def _finalize():
        m1, i1 = _half_min(best_ref[...], brb_ref[...])
        m0 = h0v_ref[...]
        i0 = h0i_ref[...]
        m0q = m0.astype(jnp.bfloat16).astype(jnp.float32)
        take0 = jnp.logical_or(m0q < m1,
                               jnp.logical_and(m0q == m1, i0 < i1))
        idx_ref[0, 0, :] = jnp.where(take0, i0, i1)[0]
        part = jnp.sum(jnp.where(take0, m0, m1))
        tot = jnp.where(b == 0, 0.0, acc_ref[0]) + part
        acc_ref[0] = tot

        @pl.when(b == _B - 1)
        def _loss():
            loss_ref[0, 0] = tot * _LOSS_SCALE


def _dist_argmin(zm2b, z2, cbb, c2):
    return pl.pallas_call(
        _dist_argmin_body,
        grid=(_B, _KT),
        in_specs=[
            pl.BlockSpec((1, _D, _T), lambda b, kt: (b, 0, 0)),
            pl.BlockSpec((1, 1, _T), lambda b, kt: (b, 0, 0)),
            pl.BlockSpec((_TK, _D), lambda b, kt: (kt, 0)),
            pl.BlockSpec((_TK, 1), lambda b, kt: (kt, 0)),
        ],
        out_specs=[
            pl.BlockSpec((1, 1, _T), lambda b, kt: (b, 0, 0)),
            pl.BlockSpec(memory_space=pltpu.SMEM),
        ],
        out_shape=[
            jax.ShapeDtypeStruct((_B, 1, _T), jnp.int32),
            jax.ShapeDtypeStruct((1, 1), jnp.float32),
        ],
        scratch_shapes=[
            pltpu.VMEM((8, _T), jnp.float32),
            pltpu.VMEM((8, _T), jnp.int32),
            pltpu.VMEM((1, _T), jnp.float32),
            pltpu.VMEM((1, _T), jnp.int32),
            pltpu.SMEM((1,), jnp.float32),
        ],
    )(zm2b, z2, cbb, c2)


def _xpose_body(in_ref, out_ref):
    out_ref[0] = in_ref[...].T


def _xpose(zq_flat):
    return pl.pallas_call(
        _xpose_body,
        grid=(_B,),
        in_specs=[pl.BlockSpec((_T, _D), lambda b: (b, 0))],
        out_specs=pl.BlockSpec((1, _D, _T), lambda b: (b, 0, 0)),
        out_shape=jax.ShapeDtypeStruct((_B, _D, _T), jnp.float32),
    )(zq_flat)


@functools.lru_cache(maxsize=1)
def _sc_gather_kernel():
    @functools.partial(
        pl.kernel,
        out_type=jax.ShapeDtypeStruct((_NTOK, _D), jnp.float32),
        mesh=plsc.VectorSubcoreMesh(core_axis_name="c", subcore_axis_name="s",
                                    num_cores=_NC, num_subcores=_NS),
        scratch_types=[
            pltpu.VMEM((_BPW,), jnp.int32),
            pltpu.VMEM((_BPW, _D), jnp.float32),
            pltpu.SemaphoreType.DMA,
        ],
    )
    def _sc_gather(cb_hbm, idx_hbm, out_hbm, idx_v, rows_v, sem):
        wid = lax.axis_index("s") * _NC + lax.axis_index("c")
        base = wid * _BPW
        pltpu.sync_copy(idx_hbm.at[pl.ds(base, _BPW)], idx_v)
        pltpu.async_copy(cb_hbm.at[idx_v], rows_v, sem).wait()
        pltpu.sync_copy(rows_v, out_hbm.at[pl.ds(base, _BPW)])

    return _sc_gather


def kernel(z, codebook):
    zm2b = (-2.0 * z).astype(jnp.bfloat16)
    cbb = codebook.astype(jnp.bfloat16)
    z2 = jnp.sum(z ** 2, axis=1).reshape(_B, 1, _T)
    c2 = jnp.sum(codebook ** 2, axis=1).reshape(_K, 1)

    idx3, loss = _dist_argmin(zm2b, z2, cbb, c2)

    zq_flat = _sc_gather_kernel()(codebook, idx3.reshape(_NTOK))
    z_q = _xpose(zq_flat)
    return z_q, idx3.reshape(_B, _T), loss.reshape(())


# speedup vs baseline: 1.2700x; 1.0219x over previous
import functools

import jax
import jax.numpy as jnp
from jax import lax
from jax.experimental import pallas as pl
from jax.experimental.pallas import tpu as pltpu
from jax.experimental.pallas import tpu_sc as plsc

_B, _D, _T = 8, 256, 1024
_K = 8192
_TK = 4096
_KT = _K // _TK
_RB = _TK // 8
_NTOK = _B * _T
_LOSS_SCALE = 1.25 / float(_B * _D * _T)

_NC, _NS = 2, 16
_NW = _NC * _NS
_BPW = _NTOK // _NW


def _dist_argmin_body(zb_ref, z2_ref, cb_ref, c2_ref, idx_ref, loss_ref,
                      acc_ref):
    b = pl.program_id(0)

    def _half(h):
        dot = jnp.dot(cb_ref[h * 4096:(h + 1) * 4096, :], zb_ref[0],
                      preferred_element_type=jnp.float32)
        s = (z2_ref[0] + dot) + c2_ref[h * 4096:(h + 1) * 4096, :]
        base_rb = h * 512
        bestv = s[0:8, :]
        besti = jnp.full((8, _T), base_rb, dtype=jnp.int32)
        for r in range(1, 512):
            srow = s[r * 8:(r + 1) * 8, :]
            cmp = srow < bestv
            bestv = jnp.where(cmp, srow, bestv)
            besti = jnp.where(cmp, base_rb + r, besti)
        kglob = besti * 8 + lax.broadcasted_iota(jnp.int32, (8, _T), 0)
        m = jnp.min(bestv, axis=0, keepdims=True)
        cand = jnp.where(bestv == m, kglob, _K)
        return m, jnp.min(cand, axis=0, keepdims=True)

    m0, i0 = _half(0)
    m1, i1 = _half(1)
    m0q = m0.astype(jnp.bfloat16).astype(jnp.float32)
    take0 = jnp.logical_or(m0q < m1,
                           jnp.logical_and(m0q == m1, i0 < i1))
    idx_ref[0, 0, :] = jnp.where(take0, i0, i1)[0]
    part = jnp.sum(jnp.where(take0, m0, m1))
    tot = jnp.where(b == 0, 0.0, acc_ref[0]) + part
    acc_ref[0] = tot

    @pl.when(b == _B - 1)
    def _loss():
        loss_ref[0, 0] = tot * _LOSS_SCALE


def _dist_argmin(zm2b, z2, cbb, c2):
    return pl.pallas_call(
        _dist_argmin_body,
        grid=(_B,),
        in_specs=[
            pl.BlockSpec((1, _D, _T), lambda b: (b, 0, 0)),
            pl.BlockSpec((1, 1, _T), lambda b: (b, 0, 0)),
            pl.BlockSpec((_K, _D), lambda b: (0, 0)),
            pl.BlockSpec((_K, 1), lambda b: (0, 0)),
        ],
        out_specs=[
            pl.BlockSpec((1, 1, _T), lambda b: (b, 0, 0)),
            pl.BlockSpec(memory_space=pltpu.SMEM),
        ],
        out_shape=[
            jax.ShapeDtypeStruct((_B, 1, _T), jnp.int32),
            jax.ShapeDtypeStruct((1, 1), jnp.float32),
        ],
        scratch_shapes=[
            pltpu.SMEM((1,), jnp.float32),
        ],
    )(zm2b, z2, cbb, c2)


def _xpose_body(in_ref, out_ref):
    out_ref[0] = in_ref[...].T


def _xpose(zq_flat):
    return pl.pallas_call(
        _xpose_body,
        grid=(_B,),
        in_specs=[pl.BlockSpec((_T, _D), lambda b: (b, 0))],
        out_specs=pl.BlockSpec((1, _D, _T), lambda b: (b, 0, 0)),
        out_shape=jax.ShapeDtypeStruct((_B, _D, _T), jnp.float32),
    )(zq_flat)


@functools.lru_cache(maxsize=1)
def _sc_gather_kernel():
    @functools.partial(
        pl.kernel,
        out_type=jax.ShapeDtypeStruct((_NTOK, _D), jnp.float32),
        mesh=plsc.VectorSubcoreMesh(core_axis_name="c", subcore_axis_name="s",
                                    num_cores=_NC, num_subcores=_NS),
        scratch_types=[
            pltpu.VMEM((_BPW,), jnp.int32),
            pltpu.VMEM((_BPW, _D), jnp.float32),
            pltpu.SemaphoreType.DMA,
        ],
    )
    def _sc_gather(cb_hbm, idx_hbm, out_hbm, idx_v, rows_v, sem):
        wid = lax.axis_index("s") * _NC + lax.axis_index("c")
        base = wid * _BPW
        pltpu.sync_copy(idx_hbm.at[pl.ds(base, _BPW)], idx_v)
        pltpu.async_copy(cb_hbm.at[idx_v], rows_v, sem).wait()
        pltpu.sync_copy(rows_v, out_hbm.at[pl.ds(base, _BPW)])

    return _sc_gather


def kernel(z, codebook):
    zm2b = (-2.0 * z).astype(jnp.bfloat16)
    cbb = codebook.astype(jnp.bfloat16)
    z2 = jnp.sum(z ** 2, axis=1).reshape(_B, 1, _T)
    c2 = jnp.sum(codebook ** 2, axis=1).reshape(_K, 1)

    idx3, loss = _dist_argmin(zm2b, z2, cbb, c2)

    zq_flat = _sc_gather_kernel()(codebook, idx3.reshape(_NTOK))
    z_q = _xpose(zq_flat)
    return z_q, idx3.reshape(_B, _T), loss.reshape(())
